# baseline (speedup 1.0000x reference)
import functools
import jax
import jax.numpy as jnp
from jax import lax
from jax.experimental import pallas as pl
from jax.experimental.pallas import tpu as pltpu
from jax.experimental.pallas import tpu_sc as plsc

_E = 64
_K = 8
_HID = 2048
_N = 16384
_BLK = 2048
_NW = 32
_TPW = _N // _NW
_NG = _TPW // 16


def _matmul_body(x_ref, w_ref, out_ref):
    out_ref[...] = jnp.dot(x_ref[...], w_ref[...],
                           preferred_element_type=jnp.float32)


def _tc_logits(hidden_states, gate_w):
    return pl.pallas_call(
        _matmul_body,
        grid=(_N // _BLK,),
        in_specs=[
            pl.BlockSpec((_BLK, _HID), lambda i: (i, 0)),
            pl.BlockSpec((_HID, _E), lambda i: (0, 0)),
        ],
        out_specs=pl.BlockSpec((_BLK, _E), lambda i: (i, 0)),
        out_shape=jax.ShapeDtypeStruct((_N, _E), jnp.float32),
    )(hidden_states, gate_w)


@functools.partial(
    pl.kernel,
    out_type=[
        jax.ShapeDtypeStruct((_N * _K,), jnp.float32),
        jax.ShapeDtypeStruct((_N * _K,), jnp.int32),
        jax.ShapeDtypeStruct((_NW, _E * 16), jnp.float32),
        jax.ShapeDtypeStruct((_NW, _E * 16), jnp.float32),
    ],
    mesh=plsc.VectorSubcoreMesh(core_axis_name="c", subcore_axis_name="s"),
    compiler_params=pltpu.CompilerParams(needs_layout_passes=False),
    scratch_types=[
        pltpu.VMEM((_TPW * _E,), jnp.float32),
        pltpu.VMEM((_E * 16,), jnp.float32),
        pltpu.VMEM((_E * 16,), jnp.float32),
        pltpu.VMEM((_E * 16,), jnp.float32),
        pltpu.VMEM((_TPW * _K,), jnp.float32),
        pltpu.VMEM((_TPW * _K,), jnp.int32),
    ],
)
def _sc_route(logits_hbm, w_hbm, i_hbm, cnt_hbm, psum_hbm,
              chunk, trd, accp, cnt, ow, oi):
    wid = lax.axis_index("s") * 2 + lax.axis_index("c")
    base = wid * _TPW
    pltpu.sync_copy(logits_hbm.at[pl.ds(base * _E, _TPW * _E)], chunk)

    zero = jnp.zeros((16,), jnp.float32)
    for e in range(_E):
        accp[pl.ds(e * 16, 16)] = zero
        cnt[pl.ds(e * 16, 16)] = zero
    lanes = lax.iota(jnp.int32, 16)
    ones = jnp.ones((16,), jnp.float32)

    def group(g, carry):
        rowv = g * 16 + lanes
        m = jnp.full((16,), -jnp.inf, jnp.float32)
        for e in range(_E):
            v = plsc.load_gather(chunk, [rowv * _E + e])
            trd[pl.ds(e * 16, 16)] = v
            m = jnp.maximum(m, v)
        s = jnp.zeros((16,), jnp.float32)
        ws = [jnp.full((16,), -1.0, jnp.float32) for _ in range(_K)]
        ids = [jnp.zeros((16,), jnp.int32) for _ in range(_K)]
        for e in range(_E):
            d = jnp.exp(trd[pl.ds(e * 16, 16)] - m)
            trd[pl.ds(e * 16, 16)] = d
            s = s + d
            cv = d
            ci = jnp.full((16,), e, jnp.int32)
            for k in range(_K):
                gt = cv > ws[k]
                nw = jnp.where(gt, cv, ws[k])
                ncv = jnp.where(gt, ws[k], cv)
                ni = jnp.where(gt, ci, ids[k])
                nci = jnp.where(gt, ids[k], ci)
                ws[k], ids[k] = nw, ni
                cv, ci = ncv, nci
        r = 1.0 / s
        for e in range(_E):
            sl = pl.ds(e * 16, 16)
            accp[sl] = accp[sl] + trd[sl] * r
        wsum = ws[0]
        for k in range(1, _K):
            wsum = wsum + ws[k]
        rw = 1.0 / wsum
        for k in range(_K):
            plsc.store_scatter(ow, [rowv * _K + k], ws[k] * rw)
            plsc.store_scatter(oi, [rowv * _K + k], ids[k])
            plsc.addupdate_scatter(cnt, [ids[k] * 16 + lanes], ones)
        return carry

    lax.fori_loop(0, _NG, group, 0)

    pltpu.sync_copy(ow, w_hbm.at[pl.ds(base * _K, _TPW * _K)])
    pltpu.sync_copy(oi, i_hbm.at[pl.ds(base * _K, _TPW * _K)])
    pltpu.sync_copy(cnt, cnt_hbm.at[wid])
    pltpu.sync_copy(accp, psum_hbm.at[wid])


def kernel(hidden_states, gate_w):
    logits = _tc_logits(hidden_states, gate_w)
    wout, iout, cnt_parts, psum_parts = _sc_route(logits.reshape(_N * _E))
    tokens_per_expert = cnt_parts.reshape(_NW, _E, 16).sum(axis=(0, 2))
    psum = psum_parts.reshape(_NW, _E, 16).sum(axis=(0, 2))
    f = tokens_per_expert / (_N * _K)
    p = psum / _N
    aux = _E * jnp.sum(f * p)
    return (wout.reshape(_N, _K), iout.reshape(_N, _K), aux)


# speedup vs baseline: 3.2422x; 3.2422x over previous
import jax
import jax.numpy as jnp
from jax import lax
from jax.experimental import pallas as pl
from jax.experimental.pallas import tpu as pltpu

_NUM_EXPERTS = 64
_TOP_K = 8
_HIDDEN = 2048
_NUM_TOKENS = 16384
_BLK = 2048


def _router_body(x0_ref, x1_ref, wt_ref, wout_ref, iout_ref, aux_ref,
                 cnt_ref, psum_ref):
    i = pl.program_id(0)
    nblocks = pl.num_programs(0)

    logits = jnp.concatenate(
        [lax.dot_general(
            wt_ref[...], x_ref[...],
            dimension_numbers=(((1,), (1,)), ((), ())),
            preferred_element_type=jnp.float32)
         for x_ref in (x0_ref, x1_ref)], axis=1)
    m = jnp.max(logits, axis=0, keepdims=True)
    e = jnp.exp(logits - m)
    s = jnp.sum(e, axis=0, keepdims=True)

    eidx = lax.broadcasted_iota(jnp.int32, (_NUM_EXPERTS, _BLK), 0)

    cur = e
    sel = jnp.zeros((_NUM_EXPERTS, _BLK), jnp.float32)
    vals = []
    idxs = []
    for _ in range(_TOP_K):
        mx = jnp.max(cur, axis=0, keepdims=True)
        hit = cur == mx
        amx = jnp.min(jnp.where(hit, eidx, _NUM_EXPERTS),
                      axis=0, keepdims=True)
        pick = eidx == amx
        vals.append(mx)
        idxs.append(amx)
        sel = jnp.where(pick, 1.0, sel)
        cur = jnp.where(pick, -1.0, cur)

    w8 = jnp.concatenate(vals, axis=0)
    i8 = jnp.concatenate(idxs, axis=0)
    wout_ref[...] = w8 / jnp.sum(w8, axis=0, keepdims=True)
    iout_ref[...] = i8

    probs = e * (1.0 / s)

    @pl.when(i == 0)
    def _init():
        cnt_ref[...] = jnp.zeros_like(cnt_ref)
        psum_ref[...] = jnp.zeros_like(psum_ref)
        aux_ref[...] = jnp.zeros((1, 1), jnp.float32)

    cnt_ref[...] += jnp.sum(sel, axis=1, keepdims=True)
    psum_ref[...] += jnp.sum(probs, axis=1, keepdims=True)

    @pl.when(i == nblocks - 1)
    def _fin():
        f = cnt_ref[...] / (_NUM_TOKENS * _TOP_K)
        p = psum_ref[...] / _NUM_TOKENS
        aux_ref[...] = _NUM_EXPERTS * jnp.sum(f * p, keepdims=True).reshape(1, 1)


def kernel(hidden_states, gate_w):
    nblocks = _NUM_TOKENS // _BLK
    wt = gate_w.T
    wout_t, iout_t, aux = pl.pallas_call(
        _router_body,
        grid=(nblocks,),
        in_specs=[
            pl.BlockSpec((_BLK // 2, _HIDDEN), lambda i: (2 * i, 0)),
            pl.BlockSpec((_BLK // 2, _HIDDEN), lambda i: (2 * i + 1, 0)),
            pl.BlockSpec((_NUM_EXPERTS, _HIDDEN), lambda i: (0, 0)),
        ],
        out_specs=[
            pl.BlockSpec((_TOP_K, _BLK), lambda i: (0, i)),
            pl.BlockSpec((_TOP_K, _BLK), lambda i: (0, i)),
            pl.BlockSpec((1, 1), lambda i: (0, 0)),
        ],
        out_shape=[
            jax.ShapeDtypeStruct((_TOP_K, _NUM_TOKENS), jnp.float32),
            jax.ShapeDtypeStruct((_TOP_K, _NUM_TOKENS), jnp.int32),
            jax.ShapeDtypeStruct((1, 1), jnp.float32),
        ],
        scratch_shapes=[
            pltpu.VMEM((_NUM_EXPERTS, 1), jnp.float32),
            pltpu.VMEM((_NUM_EXPERTS, 1), jnp.float32),
        ],
    )(hidden_states, hidden_states, wt)
    return (wout_t.T, iout_t.T, aux[0, 0])
